# baseline (speedup 1.0000x reference)
import functools

import jax
import jax.numpy as jnp
from jax import lax
from jax.experimental import pallas as pl
from jax.experimental.pallas import tpu as pltpu
from jax.experimental.pallas import tpu_sc as plsc

B = 4
H = 512
W = 512
C = 64
N = 120000

NC, NS, L = 2, 16, 16
NW = NC * NS
NCELLS = B * H * W
HALF = NCELLS // 2
VH = HALF // NW
PPW = VH // 2
NPAIR = HALF // 2
SEGP = 4096
CH = 1536
NP = 122880
NCHUNK = NP // CH
GPC = CH // L
CH2 = 256
CPAD = NW * CH2

_MESH = plsc.VectorSubcoreMesh(core_axis_name="c", subcore_axis_name="s",
                               num_cores=NC, num_subcores=NS)
_CPARAMS = pltpu.CompilerParams(needs_layout_passes=False)


def _pair_phase2(wid, read_pair, nseg, nlist0, nlist1, plist,
                 n0idx, n1idx, pidx, buf0, buf1,
                 feat_hbm, canvas_hbm, semg, semh, sems):
    iota = lax.iota(jnp.int32, L)

    def seg_body(si, carry):
        def prefill(g, c2):
            bb = g * L
            spread = (wid * CH2 + (bb & (CH2 - 1))) + iota
            nlist0[pl.ds(bb, L)] = spread
            nlist1[pl.ds(bb, L)] = spread
            plist[pl.ds(bb, L)] = (NPAIR + wid * CH2 + (bb & (CH2 - 1))) + iota
            return c2

        lax.fori_loop(0, SEGP // L, prefill, 0, unroll=8)

        def compact(g, cnt):
            w0, w1, pids = read_pair(si, g)
            m = (w0 >= 0) | (w1 >= 0)
            n0 = jnp.maximum(w0, 0)
            n1 = jnp.maximum(w1, 0)
            plsc.store_compressed(nlist0.at[pl.ds(cnt, L)], n0, mask=m)
            plsc.store_compressed(nlist1.at[pl.ds(cnt, L)], n1, mask=m)
            plsc.store_compressed(plist.at[pl.ds(cnt, L)], pids, mask=m)
            return cnt + jnp.sum(m.astype(jnp.int32))

        cnt = lax.fori_loop(0, SEGP // L, compact, 0, unroll=4)

        nchunks = (cnt + CH2 - 1) // CH2

        def dma_chunk(j, c2):
            def cpy(gg, c3):
                n0idx[pl.ds(gg * L, L)] = nlist0[pl.ds(j * CH2 + gg * L, L)]
                n1idx[pl.ds(gg * L, L)] = nlist1[pl.ds(j * CH2 + gg * L, L)]
                pidx[pl.ds(gg * L, L)] = plist[pl.ds(j * CH2 + gg * L, L)]
                return c3

            lax.fori_loop(0, CH2 // L, cpy, 0, unroll=8)
            g0 = pltpu.async_copy(feat_hbm.at[n0idx], buf0, semg)
            g1 = pltpu.async_copy(feat_hbm.at[n1idx], buf1, semh)
            g0.wait()
            g1.wait()

            def pack(r, c3):
                def quarter(q, c4):
                    buf0[r, pl.ds(C + q * L, L)] = buf1[r, pl.ds(q * L, L)]
                    return c4

                lax.fori_loop(0, C // L, quarter, 0, unroll=4)
                return c3

            lax.fori_loop(0, CH2, pack, 0, unroll=4)
            pltpu.async_copy(buf0, canvas_hbm.at[pidx], sems).wait()
            return c2

        lax.fori_loop(0, nchunks, dma_chunk, 0)
        return carry

    lax.fori_loop(0, nseg, seg_body, 0)


def _sc_a_body(feat_hbm, coords_hbm, canvas_hbm, wa0_hbm, wa1_hbm,
               wb0_hbm, wb1_hbm,
               winner_v, cbuf, nlist0, nlist1, plist, n0idx, n1idx, pidx,
               buf0, buf1, semc0, semc1, semg0, semh0, sems0):
    wid = lax.axis_index("s") * NC + lax.axis_index("c")
    lo_a = wid * VH
    lo_b = HALF + wid * VH
    iota = lax.iota(jnp.int32, L)
    semc = [semc0, semc1]

    neg1 = jnp.full((L,), -1, jnp.int32)

    def init_body(i, carry):
        winner_v[pl.ds(i * L, L)] = neg1
        return carry

    lax.fori_loop(0, 2 * VH // L, init_body, 0, unroll=8)

    def issue_coords(ci, slot):
        off = ci * CH
        return pltpu.async_copy(coords_hbm.at[:, pl.ds(off, CH)],
                                cbuf.at[slot], semc[slot])

    issue_coords(0, 0)

    def process_chunk(ci, slot):
        @pl.when(ci + 1 < NCHUNK)
        def _():
            issue_coords(ci + 1, 1 - slot)

        pltpu.make_async_copy(coords_hbm.at[:, pl.ds(ci * CH, CH)],
                              cbuf.at[slot], semc[slot]).wait()
        off = ci * CH

        def grp(g, c2):
            bv = cbuf[slot, 0, pl.ds(g * L, L)]
            yv = cbuf[slot, 1, pl.ds(g * L, L)]
            xv = cbuf[slot, 2, pl.ds(g * L, L)]
            f = (bv & (B - 1)) * (H * W) + yv * W + xv
            nv = (off + g * L) + iota
            in_a = (f >= lo_a) & (f < lo_a + VH)
            in_b = (f >= lo_b) & (f < lo_b + VH)
            m = (in_a | in_b) & (nv < N)
            fl = jnp.where(in_a, f - lo_a, (f - lo_b) + VH) & (2 * VH - 1)
            plsc.store_scatter(winner_v, [fl], nv, mask=m)
            return c2

        lax.fori_loop(0, GPC, grp, 0, unroll=5)

    def chunk_pair(ci2, carry):
        process_chunk(2 * ci2, 0)
        process_chunk(2 * ci2 + 1, 1)
        return carry

    lax.fori_loop(0, NCHUNK // 2, chunk_pair, 0)

    def read_pair_a(si, g):
        o = si * 8192 + g * L
        w0 = winner_v[pl.ds(o, L)]
        w1 = winner_v[pl.ds(o + 4096, L)]
        pids = (wid * PPW + si * SEGP + g * L) + iota
        return w0, w1, pids

    _pair_phase2(wid, read_pair_a, PPW // SEGP, nlist0, nlist1, plist,
                 n0idx, n1idx, pidx, buf0, buf1,
                 feat_hbm, canvas_hbm, semg0, semh0, sems0)

    o = wid * PPW
    pltpu.sync_copy(winner_v.at[pl.ds(0, 4096)], wa0_hbm.at[pl.ds(o, 4096)])
    pltpu.sync_copy(winner_v.at[pl.ds(4096, 4096)],
                    wa1_hbm.at[pl.ds(o, 4096)])
    pltpu.sync_copy(winner_v.at[pl.ds(8192, 4096)],
                    wa0_hbm.at[pl.ds(o + 4096, 4096)])
    pltpu.sync_copy(winner_v.at[pl.ds(12288, 4096)],
                    wa1_hbm.at[pl.ds(o + 4096, 4096)])
    pltpu.sync_copy(winner_v.at[pl.ds(16384, 4096)],
                    wb0_hbm.at[pl.ds(o, 4096)])
    pltpu.sync_copy(winner_v.at[pl.ds(20480, 4096)],
                    wb1_hbm.at[pl.ds(o, 4096)])
    pltpu.sync_copy(winner_v.at[pl.ds(24576, 4096)],
                    wb0_hbm.at[pl.ds(o + 4096, 4096)])
    pltpu.sync_copy(winner_v.at[pl.ds(28672, 4096)],
                    wb1_hbm.at[pl.ds(o + 4096, 4096)])


_sc_a = functools.partial(
    pl.kernel,
    out_type=[
        jax.ShapeDtypeStruct((NPAIR + CPAD, 2 * C), jnp.float32),
        jax.ShapeDtypeStruct((NPAIR,), jnp.int32),
        jax.ShapeDtypeStruct((NPAIR,), jnp.int32),
        jax.ShapeDtypeStruct((NPAIR,), jnp.int32),
        jax.ShapeDtypeStruct((NPAIR,), jnp.int32),
    ],
    mesh=_MESH,
    compiler_params=_CPARAMS,
    scratch_types=[
        pltpu.VMEM((2 * VH,), jnp.int32),
        pltpu.VMEM((2, 3, CH), jnp.int32),
        pltpu.VMEM((SEGP,), jnp.int32),
        pltpu.VMEM((SEGP,), jnp.int32),
        pltpu.VMEM((SEGP,), jnp.int32),
        pltpu.VMEM((CH2,), jnp.int32),
        pltpu.VMEM((CH2,), jnp.int32),
        pltpu.VMEM((CH2,), jnp.int32),
        pltpu.VMEM((CH2, 2 * C), jnp.float32),
        pltpu.VMEM((CH2, 2 * C), jnp.float32),
        pltpu.SemaphoreType.DMA,
        pltpu.SemaphoreType.DMA,
        pltpu.SemaphoreType.DMA,
        pltpu.SemaphoreType.DMA,
        pltpu.SemaphoreType.DMA,
    ],
    name="sc_scatter_a",
)(_sc_a_body)


def _sc_b_body(feat_hbm, wb0_hbm, wb1_hbm, canvas_hbm,
               wv0, wv1, nlist0, nlist1, plist, n0idx, n1idx, pidx,
               buf0, buf1, semg0, semh0, sems0):
    wid = lax.axis_index("s") * NC + lax.axis_index("c")
    iota = lax.iota(jnp.int32, L)
    pltpu.sync_copy(wb0_hbm.at[pl.ds(wid * PPW, PPW)], wv0)
    pltpu.sync_copy(wb1_hbm.at[pl.ds(wid * PPW, PPW)], wv1)

    def read_pair_b(si, g):
        o = si * SEGP + g * L
        pids = (wid * PPW + o) + iota
        return wv0[pl.ds(o, L)], wv1[pl.ds(o, L)], pids

    _pair_phase2(wid, read_pair_b, PPW // SEGP, nlist0, nlist1, plist,
                 n0idx, n1idx, pidx, buf0, buf1,
                 feat_hbm, canvas_hbm, semg0, semh0, sems0)


_sc_b = functools.partial(
    pl.kernel,
    out_type=[
        jax.ShapeDtypeStruct((NPAIR + CPAD, 2 * C), jnp.float32),
    ],
    mesh=_MESH,
    compiler_params=_CPARAMS,
    scratch_types=[
        pltpu.VMEM((PPW,), jnp.int32),
        pltpu.VMEM((PPW,), jnp.int32),
        pltpu.VMEM((SEGP,), jnp.int32),
        pltpu.VMEM((SEGP,), jnp.int32),
        pltpu.VMEM((SEGP,), jnp.int32),
        pltpu.VMEM((CH2,), jnp.int32),
        pltpu.VMEM((CH2,), jnp.int32),
        pltpu.VMEM((CH2,), jnp.int32),
        pltpu.VMEM((CH2, 2 * C), jnp.float32),
        pltpu.VMEM((CH2, 2 * C), jnp.float32),
        pltpu.SemaphoreType.DMA,
        pltpu.SemaphoreType.DMA,
        pltpu.SemaphoreType.DMA,
    ],
    name="sc_scatter_b",
)(_sc_b_body)


HB = 16
PB = HB * W // 2


def _tc_transpose_body_first(c_ref, w0_ref, w1_ref, o_ref):
    x = c_ref[...]
    eye = (lax.broadcasted_iota(jnp.int32, (2 * C, 2 * C), 0)
           == lax.broadcasted_iota(jnp.int32, (2 * C, 2 * C), 1)
           ).astype(jnp.float32)
    y = lax.dot_general(eye, x, (((1,), (1,)), ((), ())),
                        preferred_element_type=jnp.float32,
                        precision=lax.Precision.DEFAULT)
    w0 = w0_ref[...].reshape(1, PB)
    w1 = w1_ref[...].reshape(1, PB)
    top = jnp.where(w0 >= 0, y[:C, :], 0.0).reshape(C, HB // 2, W)
    bot = jnp.where(w1 >= 0, y[C:, :], 0.0).reshape(C, HB // 2, W)
    o_ref[...] = jnp.concatenate([top, bot], axis=1).reshape(1, C, HB, W)


def _tc_transpose_body_second(c_ref, w0_ref, w1_ref, _prev_ref, o_ref):
    _tc_transpose_body_first(c_ref, w0_ref, w1_ref, o_ref)


def _tc_transpose(canvas, w0, w1, half, prev=None):
    grid = (2 * H // HB,)
    hblocks = H // HB

    in_specs = [
        pl.BlockSpec((PB, 2 * C), lambda g: (g, 0)),
        pl.BlockSpec((PB,), lambda g: (g,)),
        pl.BlockSpec((PB,), lambda g: (g,)),
    ]
    args = [canvas, w0, w1]
    kwargs = {}
    if prev is None:
        body = _tc_transpose_body_first
    else:
        body = _tc_transpose_body_second
        in_specs.append(pl.BlockSpec(memory_space=pl.ANY))
        args.append(prev)
        kwargs["input_output_aliases"] = {3: 0}

    return pl.pallas_call(
        body,
        grid=grid,
        in_specs=in_specs,
        out_specs=pl.BlockSpec(
            (1, C, HB, W),
            lambda g, h=half: (g // hblocks + 2 * h, 0, g % hblocks, 0)),
        out_shape=jax.ShapeDtypeStruct((B, C, H, W), jnp.float32),
        **kwargs,
    )(*args)


def kernel(pillar_features, pillar_coords, batch_size):
    del batch_size
    featpad = jnp.pad(pillar_features, ((0, 0), (0, C)))
    coords_t = jnp.pad(pillar_coords.astype(jnp.int32).T,
                       ((0, 0), (0, NP - N)))
    canvas_a, wa0, wa1, wb0, wb1 = _sc_a(featpad, coords_t)
    (canvas_b,) = _sc_b(featpad, wb0, wb1)
    out = _tc_transpose(canvas_a, wa0, wa1, 0)
    out = _tc_transpose(canvas_b, wb0, wb1, 1, prev=out)
    return out


# speedup vs baseline: 8.4785x; 8.4785x over previous
import functools

import jax
import jax.numpy as jnp
from jax import lax
from jax.experimental import pallas as pl
from jax.experimental.pallas import tpu as pltpu
from jax.experimental.pallas import tpu_sc as plsc

B = 4
H = 512
W = 512
C = 64
N = 120000

NC, NS, L = 2, 16, 16
NW = NC * NS
NCELLS = B * H * W
HALF = NCELLS // 2
VH = HALF // NW
SEG = 4096
NSEG = VH // SEG
CH = 1536
NP = 122880
NCHUNK = NP // CH
GPC = CH // L
CH2 = 512
CPAD = NW * CH2

_MESH = plsc.VectorSubcoreMesh(core_axis_name="c", subcore_axis_name="s",
                               num_cores=NC, num_subcores=NS)
_CPARAMS = pltpu.CompilerParams(needs_layout_passes=False)


def _phase2(wid, winner_v, wbase, nlist, clist, nidx, cidx, rowbuf,
            feat_hbm, canvas_hbm, semg, sems):
    iota = lax.iota(jnp.int32, L)
    llo = wid * VH

    def seg_body(si, carry):
        sbase = si * SEG

        def prefill(g, c2):
            bb = g * L
            nlist[pl.ds(bb, L)] = (wid * CH2 + (bb & (CH2 - 1))) + iota
            clist[pl.ds(bb, L)] = (HALF + wid * CH2 + (bb & (CH2 - 1))) + iota
            return c2

        lax.fori_loop(0, SEG // L, prefill, 0, unroll=8)

        def compact(g, cnt):
            w = winner_v[pl.ds(wbase + sbase + g * L, L)]
            m = w >= 0
            cells = (llo + sbase + g * L) + iota
            plsc.store_compressed(nlist.at[pl.ds(cnt, L)], w, mask=m)
            plsc.store_compressed(clist.at[pl.ds(cnt, L)], cells, mask=m)
            return cnt + jnp.sum(m.astype(jnp.int32))

        cnt = lax.fori_loop(0, SEG // L, compact, 0, unroll=4)

        nchunks = (cnt + CH2 - 1) // CH2

        def dma_chunk(j, c2):
            def cpy(gg, c3):
                nidx[pl.ds(gg * L, L)] = nlist[pl.ds(j * CH2 + gg * L, L)]
                cidx[pl.ds(gg * L, L)] = clist[pl.ds(j * CH2 + gg * L, L)]
                return c3

            lax.fori_loop(0, CH2 // L, cpy, 0, unroll=8)
            pltpu.async_copy(feat_hbm.at[nidx], rowbuf, semg).wait()
            pltpu.async_copy(rowbuf, canvas_hbm.at[cidx], sems).wait()
            return c2

        lax.fori_loop(0, nchunks, dma_chunk, 0)
        return carry

    lax.fori_loop(0, NSEG, seg_body, 0)


def _sc_a_body(feat_hbm, coords_hbm, canvas_hbm, wa_hbm, wb_hbm,
               winner_v, cbuf, nlist, clist, nidx, cidx, rowbuf,
               semc0, semc1, semg0, sems0):
    wid = lax.axis_index("s") * NC + lax.axis_index("c")
    lo_a = wid * VH
    lo_b = HALF + wid * VH
    iota = lax.iota(jnp.int32, L)
    semc = [semc0, semc1]

    neg1 = jnp.full((L,), -1, jnp.int32)

    def init_body(i, carry):
        winner_v[pl.ds(i * L, L)] = neg1
        return carry

    lax.fori_loop(0, 2 * VH // L, init_body, 0, unroll=8)

    def issue_coords(ci, slot):
        off = ci * CH
        return pltpu.async_copy(coords_hbm.at[:, pl.ds(off, CH)],
                                cbuf.at[slot], semc[slot])

    issue_coords(0, 0)

    def process_chunk(ci, slot):
        @pl.when(ci + 1 < NCHUNK)
        def _():
            issue_coords(ci + 1, 1 - slot)

        pltpu.make_async_copy(coords_hbm.at[:, pl.ds(ci * CH, CH)],
                              cbuf.at[slot], semc[slot]).wait()
        off = ci * CH

        def grp(g, c2):
            bv = cbuf[slot, 0, pl.ds(g * L, L)]
            yv = cbuf[slot, 1, pl.ds(g * L, L)]
            xv = cbuf[slot, 2, pl.ds(g * L, L)]
            f = (bv & (B - 1)) * (H * W) + yv * W + xv
            nv = (off + g * L) + iota
            in_a = (f >= lo_a) & (f < lo_a + VH)
            in_b = (f >= lo_b) & (f < lo_b + VH)
            m = (in_a | in_b) & (nv < N)
            fl = jnp.where(in_a, f - lo_a, (f - lo_b) + VH) & (2 * VH - 1)
            plsc.store_scatter(winner_v, [fl], nv, mask=m)
            return c2

        lax.fori_loop(0, GPC, grp, 0, unroll=5)

    def chunk_pair(ci2, carry):
        process_chunk(2 * ci2, 0)
        process_chunk(2 * ci2 + 1, 1)
        return carry

    lax.fori_loop(0, NCHUNK // 2, chunk_pair, 0)

    _phase2(wid, winner_v, 0, nlist, clist, nidx, cidx, rowbuf,
            feat_hbm, canvas_hbm, semg0, sems0)

    pltpu.sync_copy(winner_v.at[pl.ds(0, VH)], wa_hbm.at[pl.ds(wid * VH, VH)])
    pltpu.sync_copy(winner_v.at[pl.ds(VH, VH)], wb_hbm.at[pl.ds(wid * VH, VH)])


_sc_a = functools.partial(
    pl.kernel,
    out_type=[
        jax.ShapeDtypeStruct((HALF + CPAD, 2 * C), jnp.float32),
        jax.ShapeDtypeStruct((HALF,), jnp.int32),
        jax.ShapeDtypeStruct((HALF,), jnp.int32),
    ],
    mesh=_MESH,
    compiler_params=_CPARAMS,
    scratch_types=[
        pltpu.VMEM((2 * VH,), jnp.int32),
        pltpu.VMEM((2, 3, CH), jnp.int32),
        pltpu.VMEM((SEG,), jnp.int32),
        pltpu.VMEM((SEG,), jnp.int32),
        pltpu.VMEM((CH2,), jnp.int32),
        pltpu.VMEM((CH2,), jnp.int32),
        pltpu.VMEM((CH2, 2 * C), jnp.float32),
        pltpu.SemaphoreType.DMA,
        pltpu.SemaphoreType.DMA,
        pltpu.SemaphoreType.DMA,
        pltpu.SemaphoreType.DMA,
    ],
    name="sc_scatter_a",
)(_sc_a_body)


def _sc_b_body(feat_hbm, wb_hbm, canvas_hbm,
               winner_v, nlist, clist, nidx, cidx, rowbuf, semg0, sems0):
    wid = lax.axis_index("s") * NC + lax.axis_index("c")
    pltpu.sync_copy(wb_hbm.at[pl.ds(wid * VH, VH)], winner_v)
    _phase2(wid, winner_v, 0, nlist, clist, nidx, cidx, rowbuf,
            feat_hbm, canvas_hbm, semg0, sems0)


_sc_b = functools.partial(
    pl.kernel,
    out_type=[
        jax.ShapeDtypeStruct((HALF + CPAD, 2 * C), jnp.float32),
    ],
    mesh=_MESH,
    compiler_params=_CPARAMS,
    scratch_types=[
        pltpu.VMEM((VH,), jnp.int32),
        pltpu.VMEM((SEG,), jnp.int32),
        pltpu.VMEM((SEG,), jnp.int32),
        pltpu.VMEM((CH2,), jnp.int32),
        pltpu.VMEM((CH2,), jnp.int32),
        pltpu.VMEM((CH2, 2 * C), jnp.float32),
        pltpu.SemaphoreType.DMA,
        pltpu.SemaphoreType.DMA,
    ],
    name="sc_scatter_b",
)(_sc_b_body)


HB = 16


def _tc_transpose_body_first(c_ref, w_ref, o_ref):
    x = c_ref[...]
    eye = (lax.broadcasted_iota(jnp.int32, (C, 2 * C), 0)
           == lax.broadcasted_iota(jnp.int32, (C, 2 * C), 1)).astype(jnp.float32)
    y = lax.dot_general(eye, x, (((1,), (1,)), ((), ())),
                        preferred_element_type=jnp.float32,
                        precision=lax.Precision.DEFAULT)
    wv = w_ref[...].reshape(1, HB * W)
    o_ref[...] = jnp.where(wv >= 0, y, 0.0).reshape(1, C, HB, W)


def _tc_transpose_body_second(c_ref, w_ref, _prev_ref, o_ref):
    _tc_transpose_body_first(c_ref, w_ref, o_ref)


def _tc_transpose(canvas, winner, half, prev=None):
    grid = (2 * H // HB,)
    hblocks = H // HB

    in_specs = [
        pl.BlockSpec((HB * W, 2 * C), lambda g: (g, 0)),
        pl.BlockSpec((HB * W,), lambda g: (g,)),
    ]
    args = [canvas, winner]
    kwargs = {}
    if prev is None:
        body = _tc_transpose_body_first
    else:
        body = _tc_transpose_body_second
        in_specs.append(pl.BlockSpec(memory_space=pl.ANY))
        args.append(prev)
        kwargs["input_output_aliases"] = {2: 0}

    return pl.pallas_call(
        body,
        grid=grid,
        in_specs=in_specs,
        out_specs=pl.BlockSpec(
            (1, C, HB, W),
            lambda g, h=half: (g // hblocks + 2 * h, 0, g % hblocks, 0)),
        out_shape=jax.ShapeDtypeStruct((B, C, H, W), jnp.float32),
        **kwargs,
    )(*args)


def kernel(pillar_features, pillar_coords, batch_size):
    del batch_size
    featpad = jnp.pad(pillar_features, ((0, 0), (0, C)))
    coords_t = jnp.pad(pillar_coords.astype(jnp.int32).T,
                       ((0, 0), (0, NP - N)))
    canvas_a, winner_a, winner_b = _sc_a(featpad, coords_t)
    (canvas_b,) = _sc_b(featpad, winner_b)
    out = _tc_transpose(canvas_a, winner_a, 0)
    out = _tc_transpose(canvas_b, winner_b, 1, prev=out)
    return out


# speedup vs baseline: 8.7590x; 1.0331x over previous
import functools

import jax
import jax.numpy as jnp
from jax import lax
from jax.experimental import pallas as pl
from jax.experimental.pallas import tpu as pltpu
from jax.experimental.pallas import tpu_sc as plsc

B = 4
H = 512
W = 512
C = 64
N = 120000

NC, NS, L = 2, 16, 16
NW = NC * NS
NCELLS = B * H * W
HALF = NCELLS // 2
VH = HALF // NW
SEG = 4096
NSEG = VH // SEG
CH = 1536
NP = 122880
NCHUNK = NP // CH
GPC = CH // L
CH2 = 512
CPAD = NW * CH2

_MESH = plsc.VectorSubcoreMesh(core_axis_name="c", subcore_axis_name="s",
                               num_cores=NC, num_subcores=NS)
_CPARAMS = pltpu.CompilerParams(needs_layout_passes=False)


def _phase2(wid, winner_v, wbase, nlist, clist, nidx, cidx, rowbuf,
            feat_hbm, canvas_hbm, semg, sems):
    iota = lax.iota(jnp.int32, L)
    llo = wid * VH

    def seg_body(si, carry):
        sbase = si * SEG

        def prefill(g, c2):
            bb = g * L
            nlist[pl.ds(bb, L)] = (wid * CH2 + (bb & (CH2 - 1))) + iota
            clist[pl.ds(bb, L)] = (HALF + wid * CH2 + (bb & (CH2 - 1))) + iota
            return c2

        lax.fori_loop(0, SEG // L, prefill, 0, unroll=8)

        def compact(g, cnt):
            w = winner_v[pl.ds(wbase + sbase + g * L, L)]
            m = w >= 0
            cells = (llo + sbase + g * L) + iota
            plsc.store_compressed(nlist.at[pl.ds(cnt, L)], w, mask=m)
            plsc.store_compressed(clist.at[pl.ds(cnt, L)], cells, mask=m)
            return cnt + jnp.sum(m.astype(jnp.int32))

        cnt = lax.fori_loop(0, SEG // L, compact, 0, unroll=4)

        nchunks = (cnt + CH2 - 1) // CH2

        def dma_chunk(j, c2):
            def cpy(gg, c3):
                nidx[pl.ds(gg * L, L)] = nlist[pl.ds(j * CH2 + gg * L, L)]
                cidx[pl.ds(gg * L, L)] = clist[pl.ds(j * CH2 + gg * L, L)]
                return c3

            lax.fori_loop(0, CH2 // L, cpy, 0, unroll=8)
            pltpu.async_copy(feat_hbm.at[nidx], rowbuf, semg).wait()
            pltpu.async_copy(rowbuf, canvas_hbm.at[cidx], sems).wait()
            return c2

        lax.fori_loop(0, nchunks, dma_chunk, 0)
        return carry

    lax.fori_loop(0, NSEG, seg_body, 0)


def _sc_a_body(feat_hbm, coords_hbm, canvas_hbm, wa_hbm, wb_hbm,
               winner_v, cbuf, nlist, clist, nidx, cidx, rowbuf,
               semc0, semc1, semg0, sems0):
    wid = lax.axis_index("s") * NC + lax.axis_index("c")
    lo_a = wid * VH
    lo_b = HALF + wid * VH
    iota = lax.iota(jnp.int32, L)
    semc = [semc0, semc1]

    neg1 = jnp.full((L,), -1, jnp.int32)

    def init_body(i, carry):
        winner_v[pl.ds(i * L, L)] = neg1
        return carry

    lax.fori_loop(0, 2 * VH // L, init_body, 0, unroll=8)

    def issue_coords(ci, slot):
        off = ci * CH
        return pltpu.async_copy(coords_hbm.at[:, pl.ds(off, CH)],
                                cbuf.at[slot], semc[slot])

    issue_coords(0, 0)

    def process_chunk(ci, slot):
        @pl.when(ci + 1 < NCHUNK)
        def _():
            issue_coords(ci + 1, 1 - slot)

        pltpu.make_async_copy(coords_hbm.at[:, pl.ds(ci * CH, CH)],
                              cbuf.at[slot], semc[slot]).wait()
        off = ci * CH

        def grp(g, c2):
            bv = cbuf[slot, 0, pl.ds(g * L, L)]
            yv = cbuf[slot, 1, pl.ds(g * L, L)]
            xv = cbuf[slot, 2, pl.ds(g * L, L)]
            f = (bv & (B - 1)) * (H * W) + yv * W + xv
            nv = (off + g * L) + iota
            in_a = (f >= lo_a) & (f < lo_a + VH)
            in_b = (f >= lo_b) & (f < lo_b + VH)
            m = (in_a | in_b) & (nv < N)
            fl = jnp.where(in_a, f - lo_a, (f - lo_b) + VH) & (2 * VH - 1)
            plsc.store_scatter(winner_v, [fl], nv, mask=m)
            return c2

        lax.fori_loop(0, GPC, grp, 0, unroll=5)

    def chunk_pair(ci2, carry):
        process_chunk(2 * ci2, 0)
        process_chunk(2 * ci2 + 1, 1)
        return carry

    lax.fori_loop(0, NCHUNK // 2, chunk_pair, 0)

    _phase2(wid, winner_v, 0, nlist, clist, nidx, cidx, rowbuf,
            feat_hbm, canvas_hbm, semg0, sems0)

    pltpu.sync_copy(winner_v.at[pl.ds(0, VH)], wa_hbm.at[pl.ds(wid * VH, VH)])
    pltpu.sync_copy(winner_v.at[pl.ds(VH, VH)], wb_hbm.at[pl.ds(wid * VH, VH)])


_sc_a = functools.partial(
    pl.kernel,
    out_type=[
        jax.ShapeDtypeStruct((HALF + CPAD, 2 * C), jnp.float32),
        jax.ShapeDtypeStruct((HALF,), jnp.int32),
        jax.ShapeDtypeStruct((HALF,), jnp.int32),
    ],
    mesh=_MESH,
    compiler_params=_CPARAMS,
    scratch_types=[
        pltpu.VMEM((2 * VH,), jnp.int32),
        pltpu.VMEM((2, 3, CH), jnp.int32),
        pltpu.VMEM((SEG,), jnp.int32),
        pltpu.VMEM((SEG,), jnp.int32),
        pltpu.VMEM((CH2,), jnp.int32),
        pltpu.VMEM((CH2,), jnp.int32),
        pltpu.VMEM((CH2, 2 * C), jnp.float32),
        pltpu.SemaphoreType.DMA,
        pltpu.SemaphoreType.DMA,
        pltpu.SemaphoreType.DMA,
        pltpu.SemaphoreType.DMA,
    ],
    name="sc_scatter_a",
)(_sc_a_body)


def _sc_b_body(feat_hbm, wb_hbm, canvas_hbm,
               winner_v, nlist, clist, nidx, cidx, rowbuf, semg0, sems0):
    wid = lax.axis_index("s") * NC + lax.axis_index("c")
    pltpu.sync_copy(wb_hbm.at[pl.ds(wid * VH, VH)], winner_v)
    _phase2(wid, winner_v, 0, nlist, clist, nidx, cidx, rowbuf,
            feat_hbm, canvas_hbm, semg0, sems0)


_sc_b = functools.partial(
    pl.kernel,
    out_type=[
        jax.ShapeDtypeStruct((HALF + CPAD, 2 * C), jnp.float32),
    ],
    mesh=_MESH,
    compiler_params=_CPARAMS,
    scratch_types=[
        pltpu.VMEM((VH,), jnp.int32),
        pltpu.VMEM((SEG,), jnp.int32),
        pltpu.VMEM((SEG,), jnp.int32),
        pltpu.VMEM((CH2,), jnp.int32),
        pltpu.VMEM((CH2,), jnp.int32),
        pltpu.VMEM((CH2, 2 * C), jnp.float32),
        pltpu.SemaphoreType.DMA,
        pltpu.SemaphoreType.DMA,
    ],
    name="sc_scatter_b",
)(_sc_b_body)


HB = 32


def _tc_transpose_body_first(c_ref, w_ref, o_ref):
    x = c_ref[...]
    eye = (lax.broadcasted_iota(jnp.int32, (C, 2 * C), 0)
           == lax.broadcasted_iota(jnp.int32, (C, 2 * C), 1)).astype(jnp.float32)
    y = lax.dot_general(eye, x, (((1,), (1,)), ((), ())),
                        preferred_element_type=jnp.float32,
                        precision=lax.Precision.DEFAULT)
    wv = w_ref[...].reshape(1, HB * W)
    o_ref[...] = jnp.where(wv >= 0, y, 0.0).reshape(1, C, HB, W)


def _tc_transpose_body_second(c_ref, w_ref, _prev_ref, o_ref):
    _tc_transpose_body_first(c_ref, w_ref, o_ref)


def _tc_transpose(canvas, winner, half, prev=None):
    grid = (2 * H // HB,)
    hblocks = H // HB

    in_specs = [
        pl.BlockSpec((HB * W, 2 * C), lambda g: (g, 0)),
        pl.BlockSpec((HB * W,), lambda g: (g,)),
    ]
    args = [canvas, winner]
    kwargs = {}
    if prev is None:
        body = _tc_transpose_body_first
    else:
        body = _tc_transpose_body_second
        in_specs.append(pl.BlockSpec(memory_space=pl.ANY))
        args.append(prev)
        kwargs["input_output_aliases"] = {2: 0}

    return pl.pallas_call(
        body,
        grid=grid,
        in_specs=in_specs,
        out_specs=pl.BlockSpec(
            (1, C, HB, W),
            lambda g, h=half: (g // hblocks + 2 * h, 0, g % hblocks, 0)),
        out_shape=jax.ShapeDtypeStruct((B, C, H, W), jnp.float32),
        **kwargs,
    )(*args)


def kernel(pillar_features, pillar_coords, batch_size):
    del batch_size
    featpad = jnp.pad(pillar_features, ((0, 0), (0, C)))
    coords_t = jnp.pad(pillar_coords.astype(jnp.int32).T,
                       ((0, 0), (0, NP - N)))
    canvas_a, winner_a, winner_b = _sc_a(featpad, coords_t)
    (canvas_b,) = _sc_b(featpad, winner_b)
    out = _tc_transpose(canvas_a, winner_a, 0)
    out = _tc_transpose(canvas_b, winner_b, 1, prev=out)
    return out


# speedup vs baseline: 8.8812x; 1.0140x over previous
import functools

import jax
import jax.numpy as jnp
from jax import lax
from jax.experimental import pallas as pl
from jax.experimental.pallas import tpu as pltpu
from jax.experimental.pallas import tpu_sc as plsc

B = 4
H = 512
W = 512
C = 64
N = 120000

NC, NS, L = 2, 16, 16
NW = NC * NS
NCELLS = B * H * W
HALF = NCELLS // 2
VH = HALF // NW
SEG = 4096
NSEG = VH // SEG
CH = 1536
NP = 122880
NCHUNK = NP // CH
GPC = CH // L
CH2 = 512
CPAD = NW * CH2

_MESH = plsc.VectorSubcoreMesh(core_axis_name="c", subcore_axis_name="s",
                               num_cores=NC, num_subcores=NS)
_CPARAMS = pltpu.CompilerParams(needs_layout_passes=False)


def _phase2(wid, winner_v, wbase, nlist, clist, nidx, cidx, rowbuf,
            feat_hbm, canvas_hbm, semg, sems):
    iota = lax.iota(jnp.int32, L)
    llo = wid * VH

    def seg_body(si, carry):
        sbase = si * SEG

        def prefill(g, c2):
            bb = g * L
            nlist[pl.ds(bb, L)] = (wid * CH2 + (bb & (CH2 - 1))) + iota
            clist[pl.ds(bb, L)] = (HALF + wid * CH2 + (bb & (CH2 - 1))) + iota
            return c2

        lax.fori_loop(0, SEG // L, prefill, 0, unroll=8)

        def compact(g, cnt):
            w = winner_v[pl.ds(wbase + sbase + g * L, L)]
            m = w >= 0
            cells = (llo + sbase + g * L) + iota
            plsc.store_compressed(nlist.at[pl.ds(cnt, L)], w, mask=m)
            plsc.store_compressed(clist.at[pl.ds(cnt, L)], cells, mask=m)
            return cnt + jnp.sum(m.astype(jnp.int32))

        cnt = lax.fori_loop(0, SEG // L, compact, 0, unroll=4)

        nchunks = (cnt + CH2 - 1) // CH2

        def dma_chunk(j, c2):
            def cpy(gg, c3):
                nidx[pl.ds(gg * L, L)] = nlist[pl.ds(j * CH2 + gg * L, L)]
                cidx[pl.ds(gg * L, L)] = clist[pl.ds(j * CH2 + gg * L, L)]
                return c3

            lax.fori_loop(0, CH2 // L, cpy, 0, unroll=8)
            pltpu.async_copy(feat_hbm.at[nidx], rowbuf, semg).wait()
            pltpu.async_copy(rowbuf, canvas_hbm.at[cidx], sems).wait()
            return c2

        lax.fori_loop(0, nchunks, dma_chunk, 0)
        return carry

    lax.fori_loop(0, NSEG, seg_body, 0)


def _sc_a_body(feat_hbm, coords_hbm, canvas_hbm, wa_hbm, wb_hbm,
               winner_v, cbuf, nlist, clist, nidx, cidx, rowbuf,
               semc0, semc1, semg0, sems0):
    wid = lax.axis_index("s") * NC + lax.axis_index("c")
    lo_a = wid * VH
    lo_b = HALF + wid * VH
    iota = lax.iota(jnp.int32, L)
    semc = [semc0, semc1]

    neg1 = jnp.full((L,), -1, jnp.int32)

    def init_body(i, carry):
        winner_v[pl.ds(i * L, L)] = neg1
        return carry

    lax.fori_loop(0, 2 * VH // L, init_body, 0, unroll=8)

    def issue_coords(ci, slot):
        off = ci * CH
        return pltpu.async_copy(coords_hbm.at[:, pl.ds(off, CH)],
                                cbuf.at[slot], semc[slot])

    issue_coords(0, 0)

    def process_chunk(ci, slot):
        @pl.when(ci + 1 < NCHUNK)
        def _():
            issue_coords(ci + 1, 1 - slot)

        pltpu.make_async_copy(coords_hbm.at[:, pl.ds(ci * CH, CH)],
                              cbuf.at[slot], semc[slot]).wait()
        off = ci * CH

        def grp(g, c2):
            bv = cbuf[slot, 0, pl.ds(g * L, L)]
            yv = cbuf[slot, 1, pl.ds(g * L, L)]
            xv = cbuf[slot, 2, pl.ds(g * L, L)]
            f = (bv & (B - 1)) * (H * W) + yv * W + xv
            nv = (off + g * L) + iota
            in_a = (f >= lo_a) & (f < lo_a + VH)
            in_b = (f >= lo_b) & (f < lo_b + VH)
            m = (in_a | in_b) & (nv < N)
            fl = jnp.where(in_a, f - lo_a, (f - lo_b) + VH) & (2 * VH - 1)
            plsc.store_scatter(winner_v, [fl], nv, mask=m)
            return c2

        lax.fori_loop(0, GPC, grp, 0, unroll=5)

    def chunk_pair(ci2, carry):
        process_chunk(2 * ci2, 0)
        process_chunk(2 * ci2 + 1, 1)
        return carry

    lax.fori_loop(0, NCHUNK // 2, chunk_pair, 0)

    _phase2(wid, winner_v, 0, nlist, clist, nidx, cidx, rowbuf,
            feat_hbm, canvas_hbm, semg0, sems0)

    pltpu.sync_copy(winner_v.at[pl.ds(0, VH)], wa_hbm.at[pl.ds(wid * VH, VH)])
    pltpu.sync_copy(winner_v.at[pl.ds(VH, VH)], wb_hbm.at[pl.ds(wid * VH, VH)])


_sc_a = functools.partial(
    pl.kernel,
    out_type=[
        jax.ShapeDtypeStruct((HALF + CPAD, 2 * C), jnp.float32),
        jax.ShapeDtypeStruct((HALF,), jnp.int32),
        jax.ShapeDtypeStruct((HALF,), jnp.int32),
    ],
    mesh=_MESH,
    compiler_params=_CPARAMS,
    scratch_types=[
        pltpu.VMEM((2 * VH,), jnp.int32),
        pltpu.VMEM((2, 3, CH), jnp.int32),
        pltpu.VMEM((SEG,), jnp.int32),
        pltpu.VMEM((SEG,), jnp.int32),
        pltpu.VMEM((CH2,), jnp.int32),
        pltpu.VMEM((CH2,), jnp.int32),
        pltpu.VMEM((CH2, 2 * C), jnp.float32),
        pltpu.SemaphoreType.DMA,
        pltpu.SemaphoreType.DMA,
        pltpu.SemaphoreType.DMA,
        pltpu.SemaphoreType.DMA,
    ],
    name="sc_scatter_a",
)(_sc_a_body)


def _sc_b_body(feat_hbm, wb_hbm, canvas_hbm,
               winner_v, nlist, clist, nidx, cidx, rowbuf, semg0, sems0):
    wid = lax.axis_index("s") * NC + lax.axis_index("c")
    pltpu.sync_copy(wb_hbm.at[pl.ds(wid * VH, VH)], winner_v)
    _phase2(wid, winner_v, 0, nlist, clist, nidx, cidx, rowbuf,
            feat_hbm, canvas_hbm, semg0, sems0)


_sc_b = functools.partial(
    pl.kernel,
    out_type=[
        jax.ShapeDtypeStruct((HALF + CPAD, 2 * C), jnp.float32),
    ],
    mesh=_MESH,
    compiler_params=_CPARAMS,
    scratch_types=[
        pltpu.VMEM((VH,), jnp.int32),
        pltpu.VMEM((SEG,), jnp.int32),
        pltpu.VMEM((SEG,), jnp.int32),
        pltpu.VMEM((CH2,), jnp.int32),
        pltpu.VMEM((CH2,), jnp.int32),
        pltpu.VMEM((CH2, 2 * C), jnp.float32),
        pltpu.SemaphoreType.DMA,
        pltpu.SemaphoreType.DMA,
    ],
    name="sc_scatter_b",
)(_sc_b_body)


HB = 64


def _tc_transpose_body_first(c_ref, w_ref, o_ref):
    x = c_ref[...]
    eye = (lax.broadcasted_iota(jnp.int32, (C, 2 * C), 0)
           == lax.broadcasted_iota(jnp.int32, (C, 2 * C), 1)).astype(jnp.float32)
    y = lax.dot_general(eye, x, (((1,), (1,)), ((), ())),
                        preferred_element_type=jnp.float32,
                        precision=lax.Precision.DEFAULT)
    wv = w_ref[...].reshape(1, HB * W)
    o_ref[...] = jnp.where(wv >= 0, y, 0.0).reshape(1, C, HB, W)


def _tc_transpose_body_second(c_ref, w_ref, _prev_ref, o_ref):
    _tc_transpose_body_first(c_ref, w_ref, o_ref)


def _tc_transpose(canvas, winner, half, prev=None):
    grid = (2 * H // HB,)
    hblocks = H // HB

    in_specs = [
        pl.BlockSpec((HB * W, 2 * C), lambda g: (g, 0)),
        pl.BlockSpec((HB * W,), lambda g: (g,)),
    ]
    args = [canvas, winner]
    kwargs = {}
    if prev is None:
        body = _tc_transpose_body_first
    else:
        body = _tc_transpose_body_second
        in_specs.append(pl.BlockSpec(memory_space=pl.ANY))
        args.append(prev)
        kwargs["input_output_aliases"] = {2: 0}

    return pl.pallas_call(
        body,
        grid=grid,
        in_specs=in_specs,
        out_specs=pl.BlockSpec(
            (1, C, HB, W),
            lambda g, h=half: (g // hblocks + 2 * h, 0, g % hblocks, 0)),
        out_shape=jax.ShapeDtypeStruct((B, C, H, W), jnp.float32),
        **kwargs,
    )(*args)


def kernel(pillar_features, pillar_coords, batch_size):
    del batch_size
    featpad = jnp.pad(pillar_features, ((0, 0), (0, C)))
    coords_t = jnp.pad(pillar_coords.astype(jnp.int32).T,
                       ((0, 0), (0, NP - N)))
    canvas_a, winner_a, winner_b = _sc_a(featpad, coords_t)
    (canvas_b,) = _sc_b(featpad, winner_b)
    out = _tc_transpose(canvas_a, winner_a, 0)
    out = _tc_transpose(canvas_b, winner_b, 1, prev=out)
    return out
